# baseline (speedup 1.0000x reference)
import functools
import numpy as np
import jax
import jax.numpy as jnp
from jax import lax
from jax.experimental import pallas as pl
from jax.experimental.pallas import tpu as pltpu
from jax.experimental.pallas import tpu_sc as plsc

N_NODES = 10000
N_EDGES = 160000
D_FEAT = 128
HIDD = 256
HEADS = 8
NUM_CLASS = 64

NP = 10240
E1 = N_EDGES + N_NODES
BE = 128
EP = 172032
CH = 16
BR = 256
RGRID = NP // BR
RP = NP // 16

_f32 = jnp.float32


def _tc1_body(x_ref, w1_ref, as_ref, ad_ref, h_ref, asrc_ref, adst_ref):
    c = pl.program_id(1)
    hc = jnp.dot(x_ref[...], w1_ref[...], preferred_element_type=_f32)
    hb16 = hc.astype(jnp.bfloat16)
    h_ref[0] = hb16[:, :64]
    h_ref[1] = hb16[:, 64:]

    @pl.when(c == 0)
    def _():
        asrc_ref[...] = jnp.zeros_like(asrc_ref)
        adst_ref[...] = jnp.zeros_like(adst_ref)

    asrc_ref[...] += jnp.dot(hc, as_ref[...], preferred_element_type=_f32)
    adst_ref[...] += jnp.dot(hc, ad_ref[...], preferred_element_type=_f32)


def _tc2_body(o1_ref, d1_ref, b1_ref, w2_ref, a2_ref, g_ref, al2_ref):
    c = pl.program_id(1)
    den = d1_ref[0] + d1_ref[1]
    rden = 1.0 / (den + 1e-16)
    colmask = (lax.broadcasted_iota(jnp.int32, (1, 16), 1) == c // 2)
    rcol = jnp.sum(jnp.where(colmask, rden, 0.0), axis=1, keepdims=True)
    rowmask = (lax.broadcasted_iota(jnp.int32, (CH, 1), 0) == c)
    brow = jnp.sum(jnp.where(rowmask, b1_ref[...], 0.0), axis=0,
                   keepdims=True)
    acc = jnp.concatenate([o1_ref[0], o1_ref[1]], axis=1) * rcol + brow
    h2c = jnp.where(acc > 0, acc, jnp.exp(jnp.minimum(acc, 0.0)) - 1.0)
    gc = jnp.dot(h2c, w2_ref[...], preferred_element_type=_f32)

    @pl.when(c == 0)
    def _():
        g_ref[...] = jnp.zeros_like(g_ref)
        al2_ref[...] = jnp.zeros_like(al2_ref)

    g_ref[...] += gc
    al2_ref[...] += jnp.dot(gc, a2_ref[...], preferred_element_type=_f32)


def _tc3_body(o2_ref, d2_ref, b2_ref, out_ref):
    s = o2_ref[0] + o2_ref[1]
    den = d2_ref[0] + d2_ref[1]
    rden = 1.0 / (den + 1e-16)
    out_ref[...] = jax.nn.sigmoid(s * rden[:, 0:1] + b2_ref[...][None, :])


_MESH = plsc.VectorSubcoreMesh(
    core_axis_name="c", subcore_axis_name="s", num_cores=2, num_subcores=16)


def _splat(v):
    return jnp.full((16,), v, jnp.int32)


@functools.partial(
    pl.kernel,
    out_type=[
        jax.ShapeDtypeStruct((EP, 16), _f32),
        jax.ShapeDtypeStruct((2 * NP, 16), _f32),
    ],
    mesh=_MESH,
    scratch_types=[
        pltpu.VMEM((BE,), jnp.int32),
        pltpu.VMEM((BE,), jnp.int32),
        pltpu.VMEM((BE, 16), _f32),
        pltpu.VMEM((BE, 16), _f32),
        pltpu.VMEM((BE, 16), _f32),
        pltpu.VMEM((RP, 16), _f32),
        pltpu.VMEM_SHARED((NP, 16), _f32),
        pltpu.SemaphoreType.DMA,
        pltpu.SemaphoreType.DMA,
    ],
    compiler_params=pltpu.CompilerParams(use_tc_tiling_on_sc=False),
)
def _edge_softmax(asrc_hbm, adst_hbm, src_hbm, dst_hbm, p_hbm, dpart_hbm,
                  srcv, dstv, asb, adb, pb, zb, dacc, sem1, sem2):
    cid = lax.axis_index("c")
    sid = lax.axis_index("s")

    zrow = jnp.zeros((16,), _f32)

    def zrow_body(i, _):
        zb[i, :] = zrow
        return 0

    lax.fori_loop(0, RP, zrow_body, 0)
    pltpu.sync_copy(zb, dacc.at[pl.ds(sid * RP, RP)])
    plsc.subcore_barrier()

    ecount = EP // 32
    base = cid * (EP // 2) + sid * ecount

    def blk(i, _):
        off = base + i * BE
        pltpu.sync_copy(src_hbm.at[pl.ds(off, BE)], srcv)
        pltpu.sync_copy(dst_hbm.at[pl.ds(off, BE)], dstv)
        pltpu.async_copy(asrc_hbm.at[srcv], asb, sem1).wait()
        pltpu.async_copy(adst_hbm.at[dstv], adb, sem2).wait()

        def row(r, _):
            a = asb[r, :] + adb[r, :]
            a = jnp.maximum(a, a * 0.2)
            pb[r, :] = jnp.exp(a)
            return 0

        lax.fori_loop(0, BE, row, 0)
        pltpu.sync_copy(pb, p_hbm.at[pl.ds(off, BE)])
        pltpu.sync_copy(pb, dacc.at[dstv], add=True)
        return 0

    lax.fori_loop(0, EP // 32 // BE, blk, 0)
    plsc.subcore_barrier()
    pltpu.sync_copy(dacc.at[pl.ds(sid * RP, RP)],
                    dpart_hbm.at[pl.ds(cid * NP + sid * RP, RP)])


def _make_msg(d_chunk, ch_per_sc, split_edges, n_tables, bf16_table=False):
    out_rows = (2 if split_edges else n_tables) * NP
    nvr = d_chunk // 16
    tdt = jnp.bfloat16 if bf16_table else _f32

    @functools.partial(
        pl.kernel,
        out_type=jax.ShapeDtypeStruct((out_rows, d_chunk), _f32),
        mesh=_MESH,
        scratch_types=[
            pltpu.VMEM((2, BE), jnp.int32),
            pltpu.VMEM((2, BE), jnp.int32),
            pltpu.VMEM((2, BE), jnp.int32),
            pltpu.VMEM((BE, d_chunk), tdt),
            pltpu.VMEM((BE, d_chunk), tdt),
            pltpu.VMEM((2, BE, d_chunk), _f32),
            pltpu.VMEM((2, BE, 16), _f32),
            pltpu.VMEM((RP, d_chunk), _f32),
            pltpu.VMEM_SHARED((NP, d_chunk), _f32),
            pltpu.SemaphoreType.DMA,
            pltpu.SemaphoreType.DMA,
            pltpu.SemaphoreType.DMA,
            pltpu.SemaphoreType.DMA,
        ],
        compiler_params=pltpu.CompilerParams(
            use_tc_tiling_on_sc=False, needs_layout_passes=False),
    )
    def msg(h_hbm, p_hbm, src_hbm, dst_hbm, out_hbm,
            srcv, dstv, idxv, hb0, hb1, ob, pb, zb, acc,
            sg0, sg1, ss0, ss1):
        cid = lax.axis_index("c")
        sid = lax.axis_index("s")

        zrow = jnp.zeros((16,), _f32)

        def zrow_body(i, _):
            for j in range(nvr):
                zb[i, pl.ds(j * 16, 16)] = zrow
            return 0

        lax.fori_loop(0, RP, zrow_body, 0)

        if split_edges:
            ecount = EP // 32
            ebase = cid * (EP // 2) + sid * ecount
        else:
            ecount = EP // 16
            ebase = sid * ecount
        nblk = ecount // BE
        nb2 = nblk // 2
        last_off = ebase + (nblk - 1) * BE

        for cc in range(ch_per_sc):
            if ch_per_sc > 1 or n_tables > 1:
                gchunk = cid * ch_per_sc + cc
            else:
                gchunk = 0
            row_off = gchunk * NP
            hcol = gchunk // (HIDD // d_chunk) if n_tables > 1 else 0

            pltpu.sync_copy(zb, acc.at[pl.ds(sid * RP, RP)])
            plsc.subcore_barrier()

            def load_srcp(b, off):
                pltpu.sync_copy(src_hbm.at[pl.ds(off, BE)], srcv.at[b])
                pltpu.sync_copy(p_hbm.at[pl.ds(off, BE)], pb.at[b])
                if n_tables > 1:
                    for t in range(BE // 16):
                        idxv[b, pl.ds(t * 16, 16)] = (
                            srcv[b, pl.ds(t * 16, 16)] + row_off)

            def load_dst(b, off):
                pltpu.sync_copy(dst_hbm.at[pl.ds(off, BE)], dstv.at[b])

            def gidx(b):
                return idxv.at[b] if n_tables > 1 else srcv.at[b]

            def scale(hbuf, b):
                def row(r, _):
                    w = plsc.load_gather(
                        pb.at[b], [_splat(r), _splat(hcol)])
                    if bf16_table:
                        for j in range(nvr // 2):
                            v = hbuf[r, pl.ds(j * 32, 32)]
                            lo, hi = plsc.unpack(
                                v, format=plsc.PackFormat.INTERLEAVED)
                            ob[b, r, pl.ds(j * 32, 16)] = lo * w
                            ob[b, r, pl.ds(j * 32 + 16, 16)] = hi * w
                    else:
                        for j in range(nvr):
                            ob[b, r, pl.ds(j * 16, 16)] = (
                                hbuf[r, pl.ds(j * 16, 16)] * w)
                    return 0

                lax.fori_loop(0, BE, row, 0, unroll=4)

            load_srcp(0, ebase)
            load_dst(0, ebase)
            pltpu.async_copy(h_hbm.at[gidx(0)], hb0, sg0)

            def body2(j, _):
                off1 = ebase + (2 * j + 1) * BE
                off2 = jnp.minimum(off1 + BE, last_off)

                @pl.when(j > 0)
                def _():
                    pltpu.make_async_copy(ob.at[1], acc.at[dstv.at[1]],
                                          ss1).wait()
                load_srcp(1, off1)
                load_dst(1, off1)
                pltpu.async_copy(h_hbm.at[gidx(1)], hb1, sg1)

                pltpu.make_async_copy(h_hbm.at[gidx(0)], hb0, sg0).wait()
                scale(hb0, 0)
                pltpu.async_copy(ob.at[0], acc.at[dstv.at[0]], ss0,
                                 add=True)

                load_srcp(0, off2)

                pltpu.make_async_copy(h_hbm.at[gidx(1)], hb1, sg1).wait()
                scale(hb1, 1)

                pltpu.make_async_copy(ob.at[0], acc.at[dstv.at[0]],
                                      ss0).wait()
                load_dst(0, off2)
                pltpu.async_copy(h_hbm.at[gidx(0)], hb0, sg0)

                pltpu.async_copy(ob.at[1], acc.at[dstv.at[1]], ss1,
                                 add=True)
                return 0

            lax.fori_loop(0, nb2, body2, 0)
            pltpu.make_async_copy(h_hbm.at[gidx(0)], hb0, sg0).wait()
            pltpu.make_async_copy(ob.at[1], acc.at[dstv.at[1]],
                                  ss1).wait()

            plsc.subcore_barrier()
            if split_edges:
                oslice = cid * NP + sid * RP
            else:
                oslice = row_off + sid * RP
            pltpu.sync_copy(acc.at[pl.ds(sid * RP, RP)],
                            out_hbm.at[pl.ds(oslice, RP)])
            plsc.subcore_barrier()

    return msg


_msg1 = _make_msg(d_chunk=64, ch_per_sc=16, split_edges=False, n_tables=2 * CH,
                  bf16_table=True)
_msg2 = _make_msg(d_chunk=64, ch_per_sc=1, split_edges=True, n_tables=1)

_P32 = np.concatenate([np.arange(0, 32, 2), np.arange(1, 32, 2)])
_P64 = np.concatenate([_P32, _P32 + 32])
_PERM = np.concatenate([c * 64 + _P64 for c in range(2 * CH)])


def kernel(x, edge_index, W1, a_src1, a_dst1, b1, W2, a_src2, a_dst2, b2):
    f32 = _f32
    xp = jnp.zeros((NP, D_FEAT), f32).at[:N_NODES].set(x)

    loop = jnp.arange(N_NODES, dtype=jnp.int32)
    pad = jnp.full((EP - E1,), N_NODES, jnp.int32)
    srcp = jnp.concatenate([edge_index[0], loop, pad])
    dstp = jnp.concatenate([edge_index[1], loop, pad])

    rows1 = jnp.arange(HEADS * HIDD)
    cols1 = rows1 // HIDD
    A_s1 = jnp.zeros((HEADS * HIDD, 16), f32).at[rows1, cols1].set(
        a_src1.reshape(-1))
    A_d1 = jnp.zeros((HEADS * HIDD, 16), f32).at[rows1, cols1].set(
        a_dst1.reshape(-1))
    A2 = (jnp.zeros((NUM_CLASS, 32), f32)
          .at[jnp.arange(NUM_CLASS), 0].set(a_src2[0])
          .at[jnp.arange(NUM_CLASS), 16].set(a_dst2[0]))

    h_ch, asrc1, adst1 = pl.pallas_call(
        _tc1_body,
        grid=(RGRID, CH),
        in_specs=[
            pl.BlockSpec((BR, D_FEAT), lambda r, c: (r, 0)),
            pl.BlockSpec((D_FEAT, 128), lambda r, c: (0, c)),
            pl.BlockSpec((128, 16), lambda r, c: (c, 0)),
            pl.BlockSpec((128, 16), lambda r, c: (c, 0)),
        ],
        out_specs=[
            pl.BlockSpec((2, BR, 64), lambda r, c: (c, r, 0)),
            pl.BlockSpec((BR, 16), lambda r, c: (r, 0)),
            pl.BlockSpec((BR, 16), lambda r, c: (r, 0)),
        ],
        out_shape=[
            jax.ShapeDtypeStruct((2 * CH, NP, 64), jnp.bfloat16),
            jax.ShapeDtypeStruct((NP, 16), f32),
            jax.ShapeDtypeStruct((NP, 16), f32),
        ],
        compiler_params=pltpu.CompilerParams(
            dimension_semantics=("parallel", "arbitrary")),
    )(xp, W1, A_s1, A_d1)

    p1, d1p = _edge_softmax(asrc1, adst1, srcp, dstp)
    out1 = _msg1(h_ch.reshape(2 * CH * NP, 64), p1, srcp, dstp)

    g, al2 = pl.pallas_call(
        _tc2_body,
        grid=(RGRID, CH),
        in_specs=[
            pl.BlockSpec((2, BR, 64), lambda r, c: (c, r, 0)),
            pl.BlockSpec((2, BR, 16), lambda r, c: (0, r, 0)),
            pl.BlockSpec((CH, 128), lambda r, c: (0, 0)),
            pl.BlockSpec((128, NUM_CLASS), lambda r, c: (c, 0)),
            pl.BlockSpec((NUM_CLASS, 32), lambda r, c: (0, 0)),
        ],
        out_specs=[
            pl.BlockSpec((BR, NUM_CLASS), lambda r, c: (r, 0)),
            pl.BlockSpec((BR, 32), lambda r, c: (r, 0)),
        ],
        out_shape=[
            jax.ShapeDtypeStruct((NP, NUM_CLASS), f32),
            jax.ShapeDtypeStruct((NP, 32), f32),
        ],
        compiler_params=pltpu.CompilerParams(
            dimension_semantics=("parallel", "arbitrary")),
    )(out1.reshape(2 * CH, NP, 64), d1p.reshape(2, NP, 16),
      b1[_PERM].reshape(CH, 128), W2[_PERM, :], A2)

    asrc2 = al2[:, :16]
    adst2 = al2[:, 16:]
    p2, d2p = _edge_softmax(asrc2, adst2, srcp, dstp)
    out2 = _msg2(g, p2, srcp, dstp)

    final = pl.pallas_call(
        _tc3_body,
        grid=(RGRID,),
        in_specs=[
            pl.BlockSpec((2, BR, NUM_CLASS), lambda r: (0, r, 0)),
            pl.BlockSpec((2, BR, 16), lambda r: (0, r, 0)),
            pl.BlockSpec((NUM_CLASS,), lambda r: (0,)),
        ],
        out_specs=pl.BlockSpec((BR, NUM_CLASS), lambda r: (r, 0)),
        out_shape=jax.ShapeDtypeStruct((NP, NUM_CLASS), f32),
        compiler_params=pltpu.CompilerParams(
            dimension_semantics=("arbitrary",)),
    )(out2.reshape(2, NP, NUM_CLASS), d2p.reshape(2, NP, 16), b2)

    return final[:N_NODES]


# speedup vs baseline: 1.5968x; 1.5968x over previous
import functools
import jax
import jax.numpy as jnp
from jax import lax
from jax.experimental import pallas as pl
from jax.experimental.pallas import tpu as pltpu
from jax.experimental.pallas import tpu_sc as plsc

N_NODES = 10000
N_EDGES = 160000
D_FEAT = 128
HIDD = 256
HEADS = 8
NUM_CLASS = 64

NP = 10240
E1 = N_EDGES + N_NODES
BE = 128
NSUB = 3
SB = BE * NSUB
EP = 172032
CH = 16
BR = 256
RGRID = NP // BR
RP = NP // 16

_f32 = jnp.float32


def _tc1_body(x_ref, w1_ref, as_ref, ad_ref, h_ref, asrc_ref, adst_ref):
    hc = jnp.dot(x_ref[...], w1_ref[...], preferred_element_type=_f32)
    for c in range(2 * CH):
        h_ref[c] = hc[:, c * 64:(c + 1) * 64]
    asrc_ref[...] = jnp.dot(hc, as_ref[...], preferred_element_type=_f32)
    adst_ref[...] = jnp.dot(hc, ad_ref[...], preferred_element_type=_f32)


def _tc2_body(o1_ref, d1_ref, b1_ref, w2_ref, a2_ref, g_ref, al2_ref):
    den = d1_ref[0] + d1_ref[1]
    rden = 1.0 / (den + 1e-16)
    rfull = jnp.broadcast_to(
        rden[:, :HEADS, None], (BR, HEADS, HIDD)).reshape(BR, HEADS * HIDD)
    acc = jnp.concatenate([o1_ref[c] for c in range(2 * CH)], axis=1)
    acc = acc * rfull + b1_ref[...][0][None, :]
    h2 = jnp.where(acc > 0, acc, jnp.exp(jnp.minimum(acc, 0.0)) - 1.0)
    g = jnp.dot(h2, w2_ref[...], preferred_element_type=_f32)
    g_ref[...] = g
    al2_ref[...] = jnp.dot(g, a2_ref[...], preferred_element_type=_f32)


def _tc3_body(o2_ref, d2_ref, b2_ref, out_ref):
    s = o2_ref[0] + o2_ref[1]
    den = d2_ref[0] + d2_ref[1]
    rden = 1.0 / (den + 1e-16)
    out_ref[...] = jax.nn.sigmoid(s * rden[:, 0:1] + b2_ref[...][None, :])


_MESH = plsc.VectorSubcoreMesh(
    core_axis_name="c", subcore_axis_name="s", num_cores=2, num_subcores=16)


def _splat(v):
    return jnp.full((16,), v, jnp.int32)


@functools.partial(
    pl.kernel,
    out_type=[
        jax.ShapeDtypeStruct((EP, 16), _f32),
        jax.ShapeDtypeStruct((2 * NP, 16), _f32),
    ],
    mesh=_MESH,
    scratch_types=[
        pltpu.VMEM((1, BE), jnp.int32),
        pltpu.VMEM((1, BE), jnp.int32),
        pltpu.VMEM((BE, 16), _f32),
        pltpu.VMEM((BE, 16), _f32),
        pltpu.VMEM((BE, 16), _f32),
        pltpu.VMEM((RP, 16), _f32),
        pltpu.VMEM_SHARED((NP, 16), _f32),
        pltpu.SemaphoreType.DMA,
        pltpu.SemaphoreType.DMA,
    ],
    compiler_params=pltpu.CompilerParams(use_tc_tiling_on_sc=False),
)
def _edge_softmax(asrc_hbm, adst_hbm, src_hbm, dst_hbm, p_hbm, dpart_hbm,
                  srcv, dstv, asb, adb, pb, zb, dacc, sem1, sem2):
    cid = lax.axis_index("c")
    sid = lax.axis_index("s")

    zrow = jnp.zeros((16,), _f32)

    def zrow_body(i, _):
        zb[i, :] = zrow
        return 0

    lax.fori_loop(0, RP, zrow_body, 0)
    pltpu.sync_copy(zb, dacc.at[pl.ds(sid * RP, RP)])
    plsc.subcore_barrier()

    ecount = EP // 32
    base = cid * (EP // 2) + sid * ecount

    def blk(i, _):
        off = base + i * BE
        bi = off // BE
        pltpu.sync_copy(src_hbm.at[pl.ds(bi, 1)], srcv)
        pltpu.sync_copy(dst_hbm.at[pl.ds(bi, 1)], dstv)
        pltpu.async_copy(asrc_hbm.at[srcv.at[0]], asb, sem1).wait()
        pltpu.async_copy(adst_hbm.at[dstv.at[0]], adb, sem2).wait()

        def row(r, _):
            a = asb[r, :] + adb[r, :]
            a = jnp.maximum(a, a * 0.2)
            pb[r, :] = jnp.exp(a)
            return 0

        lax.fori_loop(0, BE, row, 0, unroll=4)
        pltpu.sync_copy(pb, p_hbm.at[pl.ds(off, BE)])
        pltpu.sync_copy(pb, dacc.at[dstv.at[0]], add=True)
        return 0

    lax.fori_loop(0, EP // 32 // BE, blk, 0)
    plsc.subcore_barrier()
    pltpu.sync_copy(dacc.at[pl.ds(sid * RP, RP)],
                    dpart_hbm.at[pl.ds(cid * NP + sid * RP, RP)])


def _make_msg(d_chunk, ch_per_sc, split_edges, n_tables):
    out_rows = (2 if split_edges else n_tables) * NP
    nvr = d_chunk // 16

    @functools.partial(
        pl.kernel,
        out_type=jax.ShapeDtypeStruct((out_rows, d_chunk), _f32),
        mesh=_MESH,
        scratch_types=[
            pltpu.VMEM((2, NSUB, BE), jnp.int32),
            pltpu.VMEM((2, NSUB, BE), jnp.int32),
            pltpu.VMEM((2, NSUB, BE), jnp.int32),
            pltpu.VMEM((SB, d_chunk), _f32),
            pltpu.VMEM((SB, d_chunk), _f32),
            pltpu.VMEM((2, SB, 16), _f32),
            pltpu.VMEM((16, d_chunk), _f32),
            pltpu.VMEM_SHARED((NP, d_chunk), _f32),
            pltpu.SemaphoreType.DMA,
            pltpu.SemaphoreType.DMA,
            pltpu.SemaphoreType.DMA,
            pltpu.SemaphoreType.DMA,
            pltpu.SemaphoreType.DMA,
        ],
        compiler_params=pltpu.CompilerParams(
            use_tc_tiling_on_sc=False, needs_layout_passes=False),
    )
    def msg(h_hbm, p_hbm, src_hbm, dst_hbm, out_hbm,
            srcv, dstv, idxv, hb0, hb1, pb, zb, acc,
            sg0, sg1, ss0, ss1, sz):
        cid = lax.axis_index("c")
        sid = lax.axis_index("s")

        zrow = jnp.zeros((16,), _f32)

        def zrow_body(i, _):
            for j in range(nvr):
                zb[i, pl.ds(j * 16, 16)] = zrow
            return 0

        lax.fori_loop(0, 16, zrow_body, 0)

        def zfill():
            def zstart(q, _):
                pltpu.async_copy(
                    zb, acc.at[pl.ds(sid * RP + q * 16, 16)], sz)
                return 0

            lax.fori_loop(0, RP // 16, zstart, 0)

            def zdrain(q, _):
                pltpu.make_async_copy(
                    zb, acc.at[pl.ds(sid * RP + q * 16, 16)], sz).wait()
                return 0

            lax.fori_loop(0, RP // 16, zdrain, 0)

        if split_edges:
            ecount = EP // 32
            sbbase = (cid * (EP // 2) + sid * ecount) // BE
        else:
            ecount = EP // 16
            sbbase = (sid * ecount) // BE
        nsb = ecount // SB
        nb2 = nsb // 2
        last_sb = sbbase + (nsb - 1) * NSUB

        for cc in range(ch_per_sc):
            if ch_per_sc > 1 or n_tables > 1:
                gchunk = cid * ch_per_sc + cc
            else:
                gchunk = 0
            row_off = gchunk * NP
            hcol = gchunk // (HIDD // d_chunk) if n_tables > 1 else 0

            zfill()
            plsc.subcore_barrier()

            def load_srcp(b, sb):
                pltpu.sync_copy(src_hbm.at[pl.ds(sb, NSUB)], srcv.at[b])
                pltpu.sync_copy(p_hbm.at[pl.ds(sb * BE, SB)], pb.at[b])
                if n_tables > 1:
                    for k in range(NSUB):
                        for t in range(BE // 16):
                            idxv[b, k, pl.ds(t * 16, 16)] = (
                                srcv[b, k, pl.ds(t * 16, 16)] + row_off)

            def load_dst(b, sb):
                pltpu.sync_copy(dst_hbm.at[pl.ds(sb, NSUB)], dstv.at[b])

            def gidx(b, k):
                return idxv.at[b, k] if n_tables > 1 else srcv.at[b, k]

            def start_gather(b, hbuf, sem):
                for k in range(NSUB):
                    pltpu.async_copy(h_hbm.at[gidx(b, k)],
                                     hbuf.at[pl.ds(k * BE, BE)], sem)

            def wait_gather(b, hbuf, sem):
                for k in range(NSUB):
                    pltpu.make_async_copy(h_hbm.at[gidx(b, k)],
                                          hbuf.at[pl.ds(k * BE, BE)],
                                          sem).wait()

            def start_scatter(b, hbuf, sem):
                for k in range(NSUB):
                    pltpu.async_copy(hbuf.at[pl.ds(k * BE, BE)],
                                     acc.at[dstv.at[b, k]], sem, add=True)

            def wait_scatter(b, hbuf, sem):
                for k in range(NSUB):
                    pltpu.make_async_copy(hbuf.at[pl.ds(k * BE, BE)],
                                          acc.at[dstv.at[b, k]],
                                          sem).wait()

            def scale(hbuf, b):
                def row(r, _):
                    w = plsc.load_gather(
                        pb.at[b], [_splat(r), _splat(hcol)])
                    for j in range(nvr):
                        hbuf[r, pl.ds(j * 16, 16)] = (
                            hbuf[r, pl.ds(j * 16, 16)] * w)
                    return 0

                lax.fori_loop(0, SB, row, 0, unroll=4)

            load_srcp(0, sbbase)
            load_dst(0, sbbase)
            start_gather(0, hb0, sg0)

            def body2(j, _):
                sb1 = sbbase + (2 * j + 1) * NSUB
                sb2 = jnp.minimum(sb1 + NSUB, last_sb)

                @pl.when(j > 0)
                def _():
                    wait_scatter(1, hb1, ss1)
                load_srcp(1, sb1)
                load_dst(1, sb1)
                start_gather(1, hb1, sg1)

                wait_gather(0, hb0, sg0)
                scale(hb0, 0)
                start_scatter(0, hb0, ss0)

                load_srcp(0, sb2)

                wait_gather(1, hb1, sg1)
                scale(hb1, 1)

                wait_scatter(0, hb0, ss0)
                load_dst(0, sb2)
                start_gather(0, hb0, sg0)

                start_scatter(1, hb1, ss1)
                return 0

            lax.fori_loop(0, nb2, body2, 0)
            wait_gather(0, hb0, sg0)
            wait_scatter(1, hb1, ss1)

            plsc.subcore_barrier()
            if split_edges:
                oslice = cid * NP + sid * RP
            else:
                oslice = row_off + sid * RP
            pltpu.sync_copy(acc.at[pl.ds(sid * RP, RP)],
                            out_hbm.at[pl.ds(oslice, RP)])
            plsc.subcore_barrier()

    return msg


_msg1 = _make_msg(d_chunk=64, ch_per_sc=16, split_edges=False,
                  n_tables=2 * CH)
_msg2 = _make_msg(d_chunk=64, ch_per_sc=1, split_edges=True, n_tables=1)


def kernel(x, edge_index, W1, a_src1, a_dst1, b1, W2, a_src2, a_dst2, b2):
    f32 = _f32
    xp = jnp.zeros((NP, D_FEAT), f32).at[:N_NODES].set(x)

    loop = jnp.arange(N_NODES, dtype=jnp.int32)
    pad = jnp.full((EP - E1,), N_NODES, jnp.int32)
    srcp = jnp.concatenate([edge_index[0], loop, pad]).reshape(EP // BE, BE)
    dstp = jnp.concatenate([edge_index[1], loop, pad]).reshape(EP // BE, BE)

    rows1 = jnp.arange(HEADS * HIDD)
    cols1 = rows1 // HIDD
    A_s1 = jnp.zeros((HEADS * HIDD, 16), f32).at[rows1, cols1].set(
        a_src1.reshape(-1))
    A_d1 = jnp.zeros((HEADS * HIDD, 16), f32).at[rows1, cols1].set(
        a_dst1.reshape(-1))
    A2 = (jnp.zeros((NUM_CLASS, 32), f32)
          .at[jnp.arange(NUM_CLASS), 0].set(a_src2[0])
          .at[jnp.arange(NUM_CLASS), 16].set(a_dst2[0]))

    h_ch, asrc1, adst1 = pl.pallas_call(
        _tc1_body,
        grid=(RGRID,),
        in_specs=[
            pl.BlockSpec((BR, D_FEAT), lambda r: (r, 0)),
            pl.BlockSpec((D_FEAT, HEADS * HIDD), lambda r: (0, 0)),
            pl.BlockSpec((HEADS * HIDD, 16), lambda r: (0, 0)),
            pl.BlockSpec((HEADS * HIDD, 16), lambda r: (0, 0)),
        ],
        out_specs=[
            pl.BlockSpec((2 * CH, BR, 64), lambda r: (0, r, 0)),
            pl.BlockSpec((BR, 16), lambda r: (r, 0)),
            pl.BlockSpec((BR, 16), lambda r: (r, 0)),
        ],
        out_shape=[
            jax.ShapeDtypeStruct((2 * CH, NP, 64), f32),
            jax.ShapeDtypeStruct((NP, 16), f32),
            jax.ShapeDtypeStruct((NP, 16), f32),
        ],
        compiler_params=pltpu.CompilerParams(
            dimension_semantics=("arbitrary",)),
    )(xp, W1, A_s1, A_d1)

    p1, d1p = _edge_softmax(asrc1, adst1, srcp, dstp)
    out1 = _msg1(h_ch.reshape(2 * CH * NP, 64), p1, srcp, dstp)

    g, al2 = pl.pallas_call(
        _tc2_body,
        grid=(RGRID,),
        in_specs=[
            pl.BlockSpec((2 * CH, BR, 64), lambda r: (0, r, 0)),
            pl.BlockSpec((2, BR, 16), lambda r: (0, r, 0)),
            pl.BlockSpec((1, HEADS * HIDD), lambda r: (0, 0)),
            pl.BlockSpec((HEADS * HIDD, NUM_CLASS), lambda r: (0, 0)),
            pl.BlockSpec((NUM_CLASS, 32), lambda r: (0, 0)),
        ],
        out_specs=[
            pl.BlockSpec((BR, NUM_CLASS), lambda r: (r, 0)),
            pl.BlockSpec((BR, 32), lambda r: (r, 0)),
        ],
        out_shape=[
            jax.ShapeDtypeStruct((NP, NUM_CLASS), f32),
            jax.ShapeDtypeStruct((NP, 32), f32),
        ],
        compiler_params=pltpu.CompilerParams(
            dimension_semantics=("arbitrary",)),
    )(out1.reshape(2 * CH, NP, 64), d1p.reshape(2, NP, 16),
      b1.reshape(1, HEADS * HIDD), W2, A2)

    asrc2 = al2[:, :16]
    adst2 = al2[:, 16:]
    p2, d2p = _edge_softmax(asrc2, adst2, srcp, dstp)
    out2 = _msg2(g, p2, srcp, dstp)

    final = pl.pallas_call(
        _tc3_body,
        grid=(RGRID,),
        in_specs=[
            pl.BlockSpec((2, BR, NUM_CLASS), lambda r: (0, r, 0)),
            pl.BlockSpec((2, BR, 16), lambda r: (0, r, 0)),
            pl.BlockSpec((NUM_CLASS,), lambda r: (0,)),
        ],
        out_specs=pl.BlockSpec((BR, NUM_CLASS), lambda r: (r, 0)),
        out_shape=jax.ShapeDtypeStruct((NP, NUM_CLASS), f32),
        compiler_params=pltpu.CompilerParams(
            dimension_semantics=("arbitrary",)),
    )(out2.reshape(2, NP, NUM_CLASS), d2p.reshape(2, NP, 16), b2)

    return final[:N_NODES]


# speedup vs baseline: 1.8245x; 1.1427x over previous
import functools
import jax
import jax.numpy as jnp
from jax import lax
from jax.experimental import pallas as pl
from jax.experimental.pallas import tpu as pltpu
from jax.experimental.pallas import tpu_sc as plsc

N_NODES = 10000
N_EDGES = 160000
D_FEAT = 128
HIDD = 256
HEADS = 8
NUM_CLASS = 64

NP = 10240
E1 = N_EDGES + N_NODES
BE = 128
NSUB = 3
SB = BE * NSUB
EP = 172032
CH = 16
BR = 256
RGRID = NP // BR
RP = NP // 16

_f32 = jnp.float32


def _tc1_body(x_ref, w1_ref, as_ref, ad_ref, h_ref, asrc_ref, adst_ref):
    hc = jnp.dot(x_ref[...], w1_ref[...], preferred_element_type=_f32)
    for c in range(2 * CH):
        h_ref[c] = hc[:, c * 64:(c + 1) * 64]
    asrc_ref[...] = jnp.dot(hc, as_ref[...], preferred_element_type=_f32)
    adst_ref[...] = jnp.dot(hc, ad_ref[...], preferred_element_type=_f32)


def _tc2_body(o1_ref, d1_ref, b1_ref, w2_ref, a2_ref, g_ref, al2_ref):
    den = d1_ref[0] + d1_ref[1]
    rden = 1.0 / (den + 1e-16)
    rfull = jnp.broadcast_to(
        rden[:, :HEADS, None], (BR, HEADS, HIDD)).reshape(BR, HEADS * HIDD)
    acc = jnp.concatenate([o1_ref[c] for c in range(2 * CH)], axis=1)
    acc = acc * rfull + b1_ref[...][0][None, :]
    h2 = jnp.where(acc > 0, acc, jnp.exp(jnp.minimum(acc, 0.0)) - 1.0)
    g = jnp.dot(h2, w2_ref[...], preferred_element_type=_f32)
    g_ref[...] = g
    al2_ref[...] = jnp.dot(g, a2_ref[...], preferred_element_type=_f32)


def _tc3_body(o2_ref, d2_ref, b2_ref, out_ref):
    s = o2_ref[0] + o2_ref[1]
    den = d2_ref[0] + d2_ref[1]
    rden = 1.0 / (den + 1e-16)
    out_ref[...] = jax.nn.sigmoid(s * rden[:, 0:1] + b2_ref[...][None, :])


_MESH = plsc.VectorSubcoreMesh(
    core_axis_name="c", subcore_axis_name="s", num_cores=2, num_subcores=16)


def _splat(v):
    return jnp.full((16,), v, jnp.int32)


@functools.partial(
    pl.kernel,
    out_type=[
        jax.ShapeDtypeStruct((EP, 16), _f32),
        jax.ShapeDtypeStruct((2 * NP, 16), _f32),
    ],
    mesh=_MESH,
    scratch_types=[
        pltpu.VMEM((1, BE), jnp.int32),
        pltpu.VMEM((1, BE), jnp.int32),
        pltpu.VMEM((BE, 16), _f32),
        pltpu.VMEM((BE, 16), _f32),
        pltpu.VMEM((BE, 16), _f32),
        pltpu.VMEM((RP, 16), _f32),
        pltpu.VMEM_SHARED((NP, 16), _f32),
        pltpu.SemaphoreType.DMA,
        pltpu.SemaphoreType.DMA,
    ],
    compiler_params=pltpu.CompilerParams(use_tc_tiling_on_sc=False),
)
def _edge_softmax(asrc_hbm, adst_hbm, src_hbm, dst_hbm, p_hbm, dpart_hbm,
                  srcv, dstv, asb, adb, pb, zb, dacc, sem1, sem2):
    cid = lax.axis_index("c")
    sid = lax.axis_index("s")

    zrow = jnp.zeros((16,), _f32)

    def zrow_body(i, _):
        zb[i, :] = zrow
        return 0

    lax.fori_loop(0, RP, zrow_body, 0)
    pltpu.sync_copy(zb, dacc.at[pl.ds(sid * RP, RP)])
    plsc.subcore_barrier()

    ecount = EP // 32
    base = cid * (EP // 2) + sid * ecount

    def blk(i, _):
        off = base + i * BE
        bi = off // BE
        pltpu.sync_copy(src_hbm.at[pl.ds(bi, 1)], srcv)
        pltpu.sync_copy(dst_hbm.at[pl.ds(bi, 1)], dstv)
        pltpu.async_copy(asrc_hbm.at[srcv.at[0]], asb, sem1).wait()
        pltpu.async_copy(adst_hbm.at[dstv.at[0]], adb, sem2).wait()

        def row(r, _):
            a = asb[r, :] + adb[r, :]
            a = jnp.maximum(a, a * 0.2)
            pb[r, :] = jnp.exp(a)
            return 0

        lax.fori_loop(0, BE, row, 0, unroll=4)
        pltpu.sync_copy(pb, p_hbm.at[pl.ds(off, BE)])
        pltpu.sync_copy(pb, dacc.at[dstv.at[0]], add=True)
        return 0

    lax.fori_loop(0, EP // 32 // BE, blk, 0)
    plsc.subcore_barrier()
    pltpu.sync_copy(dacc.at[pl.ds(sid * RP, RP)],
                    dpart_hbm.at[pl.ds(cid * NP + sid * RP, RP)])


def _make_msg(d_chunk, ch_per_sc, split_edges, n_tables):
    out_rows = (2 if split_edges else n_tables) * NP
    nvr = d_chunk // 16

    @functools.partial(
        pl.kernel,
        out_type=jax.ShapeDtypeStruct((out_rows, d_chunk), _f32),
        mesh=_MESH,
        scratch_types=[
            pltpu.VMEM((2, NSUB, BE), jnp.int32),
            pltpu.VMEM((2, NSUB, BE), jnp.int32),
            pltpu.VMEM((2, NSUB, BE), jnp.int32),
            pltpu.VMEM((2, NSUB, BE), jnp.int32),
            pltpu.VMEM((SB, d_chunk), _f32),
            pltpu.VMEM((SB, d_chunk), _f32),
            pltpu.VMEM((2, SB, 16), _f32),
            pltpu.VMEM((16, d_chunk), _f32),
            pltpu.VMEM_SHARED((NP, d_chunk), _f32),
            pltpu.SemaphoreType.DMA,
            pltpu.SemaphoreType.DMA,
            pltpu.SemaphoreType.DMA,
            pltpu.SemaphoreType.DMA,
            pltpu.SemaphoreType.DMA,
            pltpu.SemaphoreType.DMA,
            pltpu.SemaphoreType.DMA,
        ],
        compiler_params=pltpu.CompilerParams(
            use_tc_tiling_on_sc=False, needs_layout_passes=False),
    )
    def msg(h_hbm, p_hbm, src_hbm, dst_hbm, out_hbm,
            srcv, dstv, idxv, dsts, hb0, hb1, pb, zb, acc,
            sg0, sg1, ss0, ss1, sz, si0, si1):
        cid = lax.axis_index("c")
        sid = lax.axis_index("s")

        zrow = jnp.zeros((16,), _f32)

        def zrow_body(i, _):
            for j in range(nvr):
                zb[i, pl.ds(j * 16, 16)] = zrow
            return 0

        lax.fori_loop(0, 16, zrow_body, 0)

        def zfill():
            def zstart(q, _):
                pltpu.async_copy(
                    zb, acc.at[pl.ds(sid * RP + q * 16, 16)], sz)
                return 0

            lax.fori_loop(0, RP // 16, zstart, 0)

            def zdrain(q, _):
                pltpu.make_async_copy(
                    zb, acc.at[pl.ds(sid * RP + q * 16, 16)], sz).wait()
                return 0

            lax.fori_loop(0, RP // 16, zdrain, 0)

        if split_edges:
            ecount = EP // 32
            sbbase = (cid * (EP // 2) + sid * ecount) // BE
        else:
            ecount = EP // 16
            sbbase = (sid * ecount) // BE
        nsb = ecount // SB
        nb2 = nsb // 2
        last_sb = sbbase + (nsb - 1) * NSUB

        def chunk_body(cc, _):
            if ch_per_sc > 1 or n_tables > 1:
                gchunk = cid * ch_per_sc + cc
            else:
                gchunk = 0
            row_off = gchunk * NP
            hcol = gchunk // (HIDD // d_chunk) if n_tables > 1 else 0

            zfill()
            plsc.subcore_barrier()

            def issue_idx(b, sb, sem):
                pltpu.async_copy(src_hbm.at[pl.ds(sb, NSUB)],
                                 srcv.at[b], sem)
                pltpu.async_copy(dst_hbm.at[pl.ds(sb, NSUB)],
                                 dstv.at[b], sem)
                pltpu.async_copy(p_hbm.at[pl.ds(sb * BE, SB)],
                                 pb.at[b], sem)

            def wait_idx(b, sb, sem):
                pltpu.make_async_copy(src_hbm.at[pl.ds(sb, NSUB)],
                                      srcv.at[b], sem).wait()
                pltpu.make_async_copy(dst_hbm.at[pl.ds(sb, NSUB)],
                                      dstv.at[b], sem).wait()
                pltpu.make_async_copy(p_hbm.at[pl.ds(sb * BE, SB)],
                                      pb.at[b], sem).wait()
                if n_tables > 1:
                    for k in range(NSUB):
                        for t in range(BE // 16):
                            idxv[b, k, pl.ds(t * 16, 16)] = (
                                srcv[b, k, pl.ds(t * 16, 16)] + row_off)

            def gidx(b, k):
                return idxv.at[b, k] if n_tables > 1 else srcv.at[b, k]

            def start_gather(b, hbuf, sem):
                for k in range(NSUB):
                    pltpu.async_copy(h_hbm.at[gidx(b, k)],
                                     hbuf.at[pl.ds(k * BE, BE)], sem)

            def wait_gather(b, hbuf, sem):
                for k in range(NSUB):
                    pltpu.make_async_copy(h_hbm.at[gidx(b, k)],
                                          hbuf.at[pl.ds(k * BE, BE)],
                                          sem).wait()

            def snap_dst(b):
                for k in range(NSUB):
                    for t in range(BE // 16):
                        dsts[b, k, pl.ds(t * 16, 16)] = (
                            dstv[b, k, pl.ds(t * 16, 16)])

            def start_scatter(b, hbuf, sem):
                for k in range(NSUB):
                    pltpu.async_copy(hbuf.at[pl.ds(k * BE, BE)],
                                     acc.at[dsts.at[b, k]], sem, add=True)

            def wait_scatter(b, hbuf, sem):
                for k in range(NSUB):
                    pltpu.make_async_copy(hbuf.at[pl.ds(k * BE, BE)],
                                          acc.at[dsts.at[b, k]],
                                          sem).wait()

            def scale(hbuf, b):
                def row(r, _):
                    w = plsc.load_gather(
                        pb.at[b], [_splat(r), _splat(hcol)])
                    for j in range(nvr):
                        hbuf[r, pl.ds(j * 16, 16)] = (
                            hbuf[r, pl.ds(j * 16, 16)] * w)
                    return 0

                lax.fori_loop(0, SB, row, 0, unroll=4)

            issue_idx(0, sbbase, si0)
            wait_idx(0, sbbase, si0)
            start_gather(0, hb0, sg0)
            issue_idx(1, sbbase + NSUB, si1)

            def body2(j, _):
                sb1 = sbbase + (2 * j + 1) * NSUB
                sb2 = jnp.minimum(sb1 + NSUB, last_sb)
                sb3 = jnp.minimum(sb2 + NSUB, last_sb)

                wait_idx(1, sb1, si1)
                start_gather(1, hb1, sg1)

                wait_gather(0, hb0, sg0)
                scale(hb0, 0)
                snap_dst(0)
                start_scatter(0, hb0, ss0)

                issue_idx(0, sb2, si0)

                @pl.when(j > 0)
                def _():
                    wait_scatter(1, hb1, ss1)

                wait_gather(1, hb1, sg1)
                scale(hb1, 1)
                snap_dst(1)
                start_scatter(1, hb1, ss1)

                wait_idx(0, sb2, si0)
                wait_scatter(0, hb0, ss0)
                start_gather(0, hb0, sg0)

                issue_idx(1, sb3, si1)
                return 0

            lax.fori_loop(0, nb2, body2, 0)
            wait_gather(0, hb0, sg0)
            wait_scatter(1, hb1, ss1)
            wait_idx(1, last_sb, si1)

            plsc.subcore_barrier()
            if split_edges:
                oslice = cid * NP + sid * RP
            else:
                oslice = row_off + sid * RP
            pltpu.sync_copy(acc.at[pl.ds(sid * RP, RP)],
                            out_hbm.at[pl.ds(oslice, RP)])
            plsc.subcore_barrier()
            return 0

        lax.fori_loop(0, ch_per_sc, chunk_body, 0)

    return msg


_msg1 = _make_msg(d_chunk=64, ch_per_sc=16, split_edges=False,
                  n_tables=2 * CH)
_msg2 = _make_msg(d_chunk=64, ch_per_sc=1, split_edges=True, n_tables=1)


def kernel(x, edge_index, W1, a_src1, a_dst1, b1, W2, a_src2, a_dst2, b2):
    f32 = _f32
    xp = jnp.zeros((NP, D_FEAT), f32).at[:N_NODES].set(x)

    loop = jnp.arange(N_NODES, dtype=jnp.int32)
    pad = jnp.full((EP - E1,), N_NODES, jnp.int32)
    srcp = jnp.concatenate([edge_index[0], loop, pad]).reshape(EP // BE, BE)
    dstp = jnp.concatenate([edge_index[1], loop, pad]).reshape(EP // BE, BE)

    rows1 = jnp.arange(HEADS * HIDD)
    cols1 = rows1 // HIDD
    A_s1 = jnp.zeros((HEADS * HIDD, 16), f32).at[rows1, cols1].set(
        a_src1.reshape(-1))
    A_d1 = jnp.zeros((HEADS * HIDD, 16), f32).at[rows1, cols1].set(
        a_dst1.reshape(-1))
    A2 = (jnp.zeros((NUM_CLASS, 32), f32)
          .at[jnp.arange(NUM_CLASS), 0].set(a_src2[0])
          .at[jnp.arange(NUM_CLASS), 16].set(a_dst2[0]))

    h_ch, asrc1, adst1 = pl.pallas_call(
        _tc1_body,
        grid=(RGRID,),
        in_specs=[
            pl.BlockSpec((BR, D_FEAT), lambda r: (r, 0)),
            pl.BlockSpec((D_FEAT, HEADS * HIDD), lambda r: (0, 0)),
            pl.BlockSpec((HEADS * HIDD, 16), lambda r: (0, 0)),
            pl.BlockSpec((HEADS * HIDD, 16), lambda r: (0, 0)),
        ],
        out_specs=[
            pl.BlockSpec((2 * CH, BR, 64), lambda r: (0, r, 0)),
            pl.BlockSpec((BR, 16), lambda r: (r, 0)),
            pl.BlockSpec((BR, 16), lambda r: (r, 0)),
        ],
        out_shape=[
            jax.ShapeDtypeStruct((2 * CH, NP, 64), f32),
            jax.ShapeDtypeStruct((NP, 16), f32),
            jax.ShapeDtypeStruct((NP, 16), f32),
        ],
        compiler_params=pltpu.CompilerParams(
            dimension_semantics=("arbitrary",)),
    )(xp, W1, A_s1, A_d1)

    p1, d1p = _edge_softmax(asrc1, adst1, srcp, dstp)
    out1 = _msg1(h_ch.reshape(2 * CH * NP, 64), p1, srcp, dstp)

    g, al2 = pl.pallas_call(
        _tc2_body,
        grid=(RGRID,),
        in_specs=[
            pl.BlockSpec((2 * CH, BR, 64), lambda r: (0, r, 0)),
            pl.BlockSpec((2, BR, 16), lambda r: (0, r, 0)),
            pl.BlockSpec((1, HEADS * HIDD), lambda r: (0, 0)),
            pl.BlockSpec((HEADS * HIDD, NUM_CLASS), lambda r: (0, 0)),
            pl.BlockSpec((NUM_CLASS, 32), lambda r: (0, 0)),
        ],
        out_specs=[
            pl.BlockSpec((BR, NUM_CLASS), lambda r: (r, 0)),
            pl.BlockSpec((BR, 32), lambda r: (r, 0)),
        ],
        out_shape=[
            jax.ShapeDtypeStruct((NP, NUM_CLASS), f32),
            jax.ShapeDtypeStruct((NP, 32), f32),
        ],
        compiler_params=pltpu.CompilerParams(
            dimension_semantics=("arbitrary",)),
    )(out1.reshape(2 * CH, NP, 64), d1p.reshape(2, NP, 16),
      b1.reshape(1, HEADS * HIDD), W2, A2)

    asrc2 = al2[:, :16]
    adst2 = al2[:, 16:]
    p2, d2p = _edge_softmax(asrc2, adst2, srcp, dstp)
    out2 = _msg2(g, p2, srcp, dstp)

    final = pl.pallas_call(
        _tc3_body,
        grid=(RGRID,),
        in_specs=[
            pl.BlockSpec((2, BR, NUM_CLASS), lambda r: (0, r, 0)),
            pl.BlockSpec((2, BR, 16), lambda r: (0, r, 0)),
            pl.BlockSpec((NUM_CLASS,), lambda r: (0,)),
        ],
        out_specs=pl.BlockSpec((BR, NUM_CLASS), lambda r: (r, 0)),
        out_shape=jax.ShapeDtypeStruct((NP, NUM_CLASS), f32),
        compiler_params=pltpu.CompilerParams(
            dimension_semantics=("arbitrary",)),
    )(out2.reshape(2, NP, NUM_CLASS), d2p.reshape(2, NP, 16), b2)

    return final[:N_NODES]


# speedup vs baseline: 1.8795x; 1.0301x over previous
import functools
import jax
import jax.numpy as jnp
from jax import lax
from jax.experimental import pallas as pl
from jax.experimental.pallas import tpu as pltpu
from jax.experimental.pallas import tpu_sc as plsc

N_NODES = 10000
N_EDGES = 160000
D_FEAT = 128
HIDD = 256
HEADS = 8
NUM_CLASS = 64

NP = 10240
E1 = N_EDGES + N_NODES
BE = 128
NSUB = 3
SB = BE * NSUB
EP = 172032
CH = 16
BR = 256
RGRID = NP // BR
RP = NP // 16

_f32 = jnp.float32


def _tc1_body(x_ref, w1_ref, as_ref, ad_ref, h_ref, asrc_ref, adst_ref):
    hc = jnp.dot(x_ref[...], w1_ref[...], preferred_element_type=_f32)
    for c in range(2 * CH):
        h_ref[c] = hc[:, c * 64:(c + 1) * 64]
    asrc_ref[...] = jnp.dot(hc, as_ref[...], preferred_element_type=_f32)
    adst_ref[...] = jnp.dot(hc, ad_ref[...], preferred_element_type=_f32)


def _tc2_body(o1_ref, d1_ref, b1_ref, w2_ref, a2_ref, g_ref, al2_ref):
    den = d1_ref[0] + d1_ref[1]
    rden = 1.0 / (den + 1e-16)
    rfull = jnp.broadcast_to(
        rden[:, :HEADS, None], (BR, HEADS, HIDD)).reshape(BR, HEADS * HIDD)
    acc = jnp.concatenate([o1_ref[c] for c in range(2 * CH)], axis=1)
    acc = acc * rfull + b1_ref[...][0][None, :]
    h2 = jnp.where(acc > 0, acc, jnp.exp(jnp.minimum(acc, 0.0)) - 1.0)
    g = jnp.dot(h2, w2_ref[...], preferred_element_type=_f32)
    g_ref[...] = g
    al2_ref[...] = jnp.dot(g, a2_ref[...], preferred_element_type=_f32)


def _tc3_body(o2_ref, d2_ref, b2_ref, out_ref):
    s = o2_ref[0] + o2_ref[1]
    den = d2_ref[0] + d2_ref[1]
    rden = 1.0 / (den + 1e-16)
    out_ref[...] = jax.nn.sigmoid(s * rden[:, 0:1] + b2_ref[...][None, :])


_MESH = plsc.VectorSubcoreMesh(
    core_axis_name="c", subcore_axis_name="s", num_cores=2, num_subcores=16)


def _splat(v):
    return jnp.full((16,), v, jnp.int32)


@functools.partial(
    pl.kernel,
    out_type=[
        jax.ShapeDtypeStruct((EP, 16), _f32),
        jax.ShapeDtypeStruct((2 * NP, 16), _f32),
    ],
    mesh=_MESH,
    scratch_types=[
        pltpu.VMEM((1, BE), jnp.int32),
        pltpu.VMEM((1, BE), jnp.int32),
        pltpu.VMEM((BE, 16), _f32),
        pltpu.VMEM((BE, 16), _f32),
        pltpu.VMEM((BE, 16), _f32),
        pltpu.VMEM((RP, 16), _f32),
        pltpu.VMEM_SHARED((NP, 16), _f32),
        pltpu.SemaphoreType.DMA,
        pltpu.SemaphoreType.DMA,
    ],
    compiler_params=pltpu.CompilerParams(use_tc_tiling_on_sc=False),
)
def _edge_softmax(asrc_hbm, adst_hbm, src_hbm, dst_hbm, p_hbm, dpart_hbm,
                  srcv, dstv, asb, adb, pb, zb, dacc, sem1, sem2):
    cid = lax.axis_index("c")
    sid = lax.axis_index("s")

    zrow = jnp.zeros((16,), _f32)

    def zrow_body(i, _):
        zb[i, :] = zrow
        return 0

    lax.fori_loop(0, RP, zrow_body, 0)
    pltpu.sync_copy(zb, dacc.at[pl.ds(sid * RP, RP)])
    plsc.subcore_barrier()

    ecount = EP // 32
    base = cid * (EP // 2) + sid * ecount

    def blk(i, _):
        off = base + i * BE
        bi = off // BE
        pltpu.async_copy(src_hbm.at[pl.ds(bi, 1)], srcv, sem1)
        pltpu.async_copy(dst_hbm.at[pl.ds(bi, 1)], dstv, sem2)
        pltpu.make_async_copy(src_hbm.at[pl.ds(bi, 1)], srcv, sem1).wait()
        pltpu.make_async_copy(dst_hbm.at[pl.ds(bi, 1)], dstv, sem2).wait()
        pltpu.async_copy(asrc_hbm.at[srcv.at[0]], asb, sem1)
        pltpu.async_copy(adst_hbm.at[dstv.at[0]], adb, sem2)
        pltpu.make_async_copy(asrc_hbm.at[srcv.at[0]], asb, sem1).wait()
        pltpu.make_async_copy(adst_hbm.at[dstv.at[0]], adb, sem2).wait()

        def row(r, _):
            a = asb[r, :] + adb[r, :]
            a = jnp.maximum(a, a * 0.2)
            pb[r, :] = jnp.exp(a)
            return 0

        lax.fori_loop(0, BE, row, 0, unroll=4)
        pltpu.sync_copy(pb, p_hbm.at[pl.ds(off, BE)])
        pltpu.sync_copy(pb, dacc.at[dstv.at[0]], add=True)
        return 0

    lax.fori_loop(0, EP // 32 // BE, blk, 0)
    plsc.subcore_barrier()
    pltpu.sync_copy(dacc.at[pl.ds(sid * RP, RP)],
                    dpart_hbm.at[pl.ds(cid * NP + sid * RP, RP)])


def _make_msg(d_chunk, ch_per_sc, split_edges, n_tables):
    out_rows = (2 if split_edges else n_tables) * NP
    nvr = d_chunk // 16

    @functools.partial(
        pl.kernel,
        out_type=jax.ShapeDtypeStruct((out_rows, d_chunk), _f32),
        mesh=_MESH,
        scratch_types=[
            pltpu.VMEM((2, NSUB, BE), jnp.int32),
            pltpu.VMEM((2, NSUB, BE), jnp.int32),
            pltpu.VMEM((2, NSUB, BE), jnp.int32),
            pltpu.VMEM((2, NSUB, BE), jnp.int32),
            pltpu.VMEM((SB, d_chunk), _f32),
            pltpu.VMEM((SB, d_chunk), _f32),
            pltpu.VMEM((2, SB, 16), _f32),
            pltpu.VMEM((16, d_chunk), _f32),
            pltpu.VMEM_SHARED((NP, d_chunk), _f32),
            pltpu.SemaphoreType.DMA,
            pltpu.SemaphoreType.DMA,
            pltpu.SemaphoreType.DMA,
            pltpu.SemaphoreType.DMA,
            pltpu.SemaphoreType.DMA,
            pltpu.SemaphoreType.DMA,
            pltpu.SemaphoreType.DMA,
        ],
        compiler_params=pltpu.CompilerParams(
            use_tc_tiling_on_sc=False, needs_layout_passes=False),
    )
    def msg(h_hbm, p_hbm, src_hbm, dst_hbm, out_hbm,
            srcv, dstv, idxv, dsts, hb0, hb1, pb, zb, acc,
            sg0, sg1, ss0, ss1, sz, si0, si1):
        cid = lax.axis_index("c")
        sid = lax.axis_index("s")

        zrow = jnp.zeros((16,), _f32)

        def zrow_body(i, _):
            for j in range(nvr):
                zb[i, pl.ds(j * 16, 16)] = zrow
            return 0

        lax.fori_loop(0, 16, zrow_body, 0)

        def zfill():
            def zstart(q, _):
                pltpu.async_copy(
                    zb, acc.at[pl.ds(sid * RP + q * 16, 16)], sz)
                return 0

            lax.fori_loop(0, RP // 16, zstart, 0)

            def zdrain(q, _):
                pltpu.make_async_copy(
                    zb, acc.at[pl.ds(sid * RP + q * 16, 16)], sz).wait()
                return 0

            lax.fori_loop(0, RP // 16, zdrain, 0)

        if split_edges:
            ecount = EP // 32
            sbbase = (cid * (EP // 2) + sid * ecount) // BE
        else:
            ecount = EP // 16
            sbbase = (sid * ecount) // BE
        nsb = ecount // SB
        nb2 = nsb // 2
        last_sb = sbbase + (nsb - 1) * NSUB

        def chunk_body(cc, _):
            if ch_per_sc > 1 or n_tables > 1:
                gchunk = cid * ch_per_sc + cc
            else:
                gchunk = 0
            row_off = gchunk * NP
            hcol = gchunk // (HIDD // d_chunk) if n_tables > 1 else 0

            zfill()
            plsc.subcore_barrier()

            def issue_idx(b, sb, sem):
                pltpu.async_copy(src_hbm.at[pl.ds(sb, NSUB)],
                                 srcv.at[b], sem)
                pltpu.async_copy(dst_hbm.at[pl.ds(sb, NSUB)],
                                 dstv.at[b], sem)
                pltpu.async_copy(p_hbm.at[pl.ds(sb * BE, SB)],
                                 pb.at[b], sem)

            def wait_idx(b, sb, sem):
                pltpu.make_async_copy(src_hbm.at[pl.ds(sb, NSUB)],
                                      srcv.at[b], sem).wait()
                pltpu.make_async_copy(dst_hbm.at[pl.ds(sb, NSUB)],
                                      dstv.at[b], sem).wait()
                pltpu.make_async_copy(p_hbm.at[pl.ds(sb * BE, SB)],
                                      pb.at[b], sem).wait()
                if n_tables > 1:
                    for k in range(NSUB):
                        for t in range(BE // 16):
                            idxv[b, k, pl.ds(t * 16, 16)] = (
                                srcv[b, k, pl.ds(t * 16, 16)] + row_off)

            def gidx(b, k):
                return idxv.at[b, k] if n_tables > 1 else srcv.at[b, k]

            def start_gather(b, hbuf, sem):
                for k in range(NSUB):
                    pltpu.async_copy(h_hbm.at[gidx(b, k)],
                                     hbuf.at[pl.ds(k * BE, BE)], sem)

            def wait_gather(b, hbuf, sem):
                for k in range(NSUB):
                    pltpu.make_async_copy(h_hbm.at[gidx(b, k)],
                                          hbuf.at[pl.ds(k * BE, BE)],
                                          sem).wait()

            def snap_dst(b):
                for k in range(NSUB):
                    for t in range(BE // 16):
                        dsts[b, k, pl.ds(t * 16, 16)] = (
                            dstv[b, k, pl.ds(t * 16, 16)])

            def start_scatter(b, hbuf, sem):
                for k in range(NSUB):
                    pltpu.async_copy(hbuf.at[pl.ds(k * BE, BE)],
                                     acc.at[dsts.at[b, k]], sem, add=True)

            def wait_scatter(b, hbuf, sem):
                for k in range(NSUB):
                    pltpu.make_async_copy(hbuf.at[pl.ds(k * BE, BE)],
                                          acc.at[dsts.at[b, k]],
                                          sem).wait()

            def scale(hbuf, b):
                def row(r, _):
                    w = plsc.load_gather(
                        pb.at[b], [_splat(r), _splat(hcol)])
                    for j in range(nvr):
                        hbuf[r, pl.ds(j * 16, 16)] = (
                            hbuf[r, pl.ds(j * 16, 16)] * w)
                    return 0

                lax.fori_loop(0, SB, row, 0, unroll=4)

            issue_idx(0, sbbase, si0)
            wait_idx(0, sbbase, si0)
            start_gather(0, hb0, sg0)
            issue_idx(1, sbbase + NSUB, si1)

            def body2(j, _):
                sb1 = sbbase + (2 * j + 1) * NSUB
                sb2 = jnp.minimum(sb1 + NSUB, last_sb)
                sb3 = jnp.minimum(sb2 + NSUB, last_sb)

                wait_idx(1, sb1, si1)
                start_gather(1, hb1, sg1)

                wait_gather(0, hb0, sg0)
                scale(hb0, 0)
                snap_dst(0)
                start_scatter(0, hb0, ss0)

                issue_idx(0, sb2, si0)

                @pl.when(j > 0)
                def _():
                    wait_scatter(1, hb1, ss1)

                wait_gather(1, hb1, sg1)
                scale(hb1, 1)
                snap_dst(1)
                start_scatter(1, hb1, ss1)

                wait_idx(0, sb2, si0)
                wait_scatter(0, hb0, ss0)
                start_gather(0, hb0, sg0)

                issue_idx(1, sb3, si1)
                return 0

            lax.fori_loop(0, nb2, body2, 0)
            wait_gather(0, hb0, sg0)
            wait_scatter(1, hb1, ss1)
            wait_idx(1, last_sb, si1)

            plsc.subcore_barrier()
            if split_edges:
                oslice = cid * NP + sid * RP
            else:
                oslice = row_off + sid * RP
            pltpu.sync_copy(acc.at[pl.ds(sid * RP, RP)],
                            out_hbm.at[pl.ds(oslice, RP)])
            plsc.subcore_barrier()
            return 0

        lax.fori_loop(0, ch_per_sc, chunk_body, 0)

    return msg


_msg1 = _make_msg(d_chunk=64, ch_per_sc=16, split_edges=False,
                  n_tables=2 * CH)
_msg2 = _make_msg(d_chunk=64, ch_per_sc=1, split_edges=True, n_tables=1)


def kernel(x, edge_index, W1, a_src1, a_dst1, b1, W2, a_src2, a_dst2, b2):
    f32 = _f32
    xp = jnp.zeros((NP, D_FEAT), f32).at[:N_NODES].set(x)

    loop = jnp.arange(N_NODES, dtype=jnp.int32)
    pad = jnp.full((EP - E1,), N_NODES, jnp.int32)
    srcp = jnp.concatenate([edge_index[0], loop, pad]).reshape(EP // BE, BE)
    dstp = jnp.concatenate([edge_index[1], loop, pad]).reshape(EP // BE, BE)

    rows1 = jnp.arange(HEADS * HIDD)
    cols1 = rows1 // HIDD
    A_s1 = jnp.zeros((HEADS * HIDD, 16), f32).at[rows1, cols1].set(
        a_src1.reshape(-1))
    A_d1 = jnp.zeros((HEADS * HIDD, 16), f32).at[rows1, cols1].set(
        a_dst1.reshape(-1))
    A2 = (jnp.zeros((NUM_CLASS, 32), f32)
          .at[jnp.arange(NUM_CLASS), 0].set(a_src2[0])
          .at[jnp.arange(NUM_CLASS), 16].set(a_dst2[0]))

    h_ch, asrc1, adst1 = pl.pallas_call(
        _tc1_body,
        grid=(RGRID,),
        in_specs=[
            pl.BlockSpec((BR, D_FEAT), lambda r: (r, 0)),
            pl.BlockSpec((D_FEAT, HEADS * HIDD), lambda r: (0, 0)),
            pl.BlockSpec((HEADS * HIDD, 16), lambda r: (0, 0)),
            pl.BlockSpec((HEADS * HIDD, 16), lambda r: (0, 0)),
        ],
        out_specs=[
            pl.BlockSpec((2 * CH, BR, 64), lambda r: (0, r, 0)),
            pl.BlockSpec((BR, 16), lambda r: (r, 0)),
            pl.BlockSpec((BR, 16), lambda r: (r, 0)),
        ],
        out_shape=[
            jax.ShapeDtypeStruct((2 * CH, NP, 64), f32),
            jax.ShapeDtypeStruct((NP, 16), f32),
            jax.ShapeDtypeStruct((NP, 16), f32),
        ],
        compiler_params=pltpu.CompilerParams(
            dimension_semantics=("arbitrary",)),
    )(xp, W1, A_s1, A_d1)

    p1, d1p = _edge_softmax(asrc1, adst1, srcp, dstp)
    out1 = _msg1(h_ch.reshape(2 * CH * NP, 64), p1, srcp, dstp)

    g, al2 = pl.pallas_call(
        _tc2_body,
        grid=(RGRID,),
        in_specs=[
            pl.BlockSpec((2 * CH, BR, 64), lambda r: (0, r, 0)),
            pl.BlockSpec((2, BR, 16), lambda r: (0, r, 0)),
            pl.BlockSpec((1, HEADS * HIDD), lambda r: (0, 0)),
            pl.BlockSpec((HEADS * HIDD, NUM_CLASS), lambda r: (0, 0)),
            pl.BlockSpec((NUM_CLASS, 32), lambda r: (0, 0)),
        ],
        out_specs=[
            pl.BlockSpec((BR, NUM_CLASS), lambda r: (r, 0)),
            pl.BlockSpec((BR, 32), lambda r: (r, 0)),
        ],
        out_shape=[
            jax.ShapeDtypeStruct((NP, NUM_CLASS), f32),
            jax.ShapeDtypeStruct((NP, 32), f32),
        ],
        compiler_params=pltpu.CompilerParams(
            dimension_semantics=("arbitrary",)),
    )(out1.reshape(2 * CH, NP, 64), d1p.reshape(2, NP, 16),
      b1.reshape(1, HEADS * HIDD), W2, A2)

    asrc2 = al2[:, :16]
    adst2 = al2[:, 16:]
    p2, d2p = _edge_softmax(asrc2, adst2, srcp, dstp)
    out2 = _msg2(g, p2, srcp, dstp)

    final = pl.pallas_call(
        _tc3_body,
        grid=(RGRID,),
        in_specs=[
            pl.BlockSpec((2, BR, NUM_CLASS), lambda r: (0, r, 0)),
            pl.BlockSpec((2, BR, 16), lambda r: (0, r, 0)),
            pl.BlockSpec((NUM_CLASS,), lambda r: (0,)),
        ],
        out_specs=pl.BlockSpec((BR, NUM_CLASS), lambda r: (r, 0)),
        out_shape=jax.ShapeDtypeStruct((NP, NUM_CLASS), f32),
        compiler_params=pltpu.CompilerParams(
            dimension_semantics=("arbitrary",)),
    )(out2.reshape(2, NP, NUM_CLASS), d2p.reshape(2, NP, 16), b2)

    return final[:N_NODES]
